# baseline (speedup 1.0000x reference)
import functools

import numpy as np
import jax
import jax.numpy as jnp
from jax import lax
from jax.experimental import pallas as pl
from jax.experimental.pallas import tpu as pltpu
from jax.experimental.pallas import tpu_sc as plsc

HIDDEN = 64
NUM_ATTR = 4
VOCAB = 300
VPAD = 304
PAIRP = VPAD * VPAD
E = 320000
N = 10000
D_NODE = 128

def _pe_const():
    p = np.arange(VOCAB, dtype=np.float64)[:, None]
    i = np.arange(0, HIDDEN, 2, dtype=np.float64)
    div = np.exp(-(i * (np.log(10000.0) / HIDDEN)))
    ang = p * div[None, :]
    pe = np.stack([np.sin(ang), np.cos(ang)], axis=-1).reshape(VOCAB, HIDDEN)
    out = np.zeros((VPAD, HIDDEN), dtype=np.float32)
    out[:VOCAB] = pe.astype(np.float32)
    return out

_PE_PAD = _pe_const()


def _tables_body(pe, e0, e1, e2, e3, w, b, out):
    w0 = w[0:64, :]
    w1 = w[64:128, :]
    w2 = w[128:192, :]
    w3 = w[192:256, :]
    ws = w0 + w1 + w2 + w3
    out[0] = jnp.dot(pe[...], ws, preferred_element_type=jnp.float32) + b[...]
    out[1] = jnp.dot(e0[...], w0, preferred_element_type=jnp.float32)
    out[2] = jnp.dot(e1[...], w1, preferred_element_type=jnp.float32)
    out[3] = jnp.dot(e2[...], w2, preferred_element_type=jnp.float32)
    out[4] = jnp.dot(e3[...], w3, preferred_element_type=jnp.float32)


def _build_tables(pe, e0, e1, e2, e3, w_edge, b_edge):
    return pl.pallas_call(
        _tables_body,
        out_shape=jax.ShapeDtypeStruct((5, VPAD, HIDDEN), jnp.float32),
    )(pe, e0, e1, e2, e3, w_edge, b_edge)


_PBLK = 8


def _pair_body(ta, tb, out):
    tpb = jnp.tile(ta[...], (1, 2))
    out[...] = tpb[:, None, :] + tb[...][None, :, :]


def _pair_table(ta, tbr):
    out = pl.pallas_call(
        _pair_body,
        grid=(VPAD // _PBLK,),
        in_specs=[
            pl.BlockSpec((_PBLK, HIDDEN), lambda i: (i, 0)),
            pl.BlockSpec((VPAD // 2, 2 * HIDDEN), lambda i: (0, 0)),
        ],
        out_specs=pl.BlockSpec((_PBLK, VPAD // 2, 2 * HIDDEN),
                               lambda i: (i, 0, 0)),
        out_shape=jax.ShapeDtypeStruct((VPAD, VPAD // 2, 2 * HIDDEN),
                                       jnp.float32),
    )(ta, tbr)
    return out.reshape(PAIRP, HIDDEN)


_TBLK = 2560


def _tr_body(a, out):
    out[...] = a[...].T


def _attr_transpose(edge_attr):
    return pl.pallas_call(
        _tr_body,
        grid=(E // _TBLK,),
        in_specs=[pl.BlockSpec((_TBLK, 5), lambda i: (i, 0))],
        out_specs=pl.BlockSpec((5, _TBLK), lambda i: (0, i)),
        out_shape=jax.ShapeDtypeStruct((5, E), jnp.int32),
    )(edge_attr)


NW = 32
EPT = E // NW
CHUNK = 80
NCHUNK = EPT // CHUNK
NBUF = 3
ORPC = CHUNK // 2


def _edge_gather_sum(u0, u1, t3, attr_t):
    mesh = plsc.VectorSubcoreMesh(core_axis_name="c", subcore_axis_name="s")

    @functools.partial(
        pl.kernel,
        out_type=(jax.ShapeDtypeStruct((E // 4, 2 * HIDDEN), jnp.float32),
                  jax.ShapeDtypeStruct((E // 4, 2 * HIDDEN), jnp.float32)),
        mesh=mesh,
        compiler_params=pltpu.CompilerParams(use_tc_tiling_on_sc=False),
        scratch_types=[
            pltpu.VMEM((5, EPT), jnp.int32),
            pltpu.VMEM((NBUF, 3, CHUNK), jnp.int32),
            pltpu.VMEM((NBUF, 3 * CHUNK, HIDDEN), jnp.float32),
            pltpu.VMEM((NBUF, ORPC, 2 * HIDDEN), jnp.float32),
            [pltpu.SemaphoreType.DMA] * NBUF,
            [pltpu.SemaphoreType.DMA] * NBUF,
        ],
    )
    def body(u0_hbm, u1_hbm, t3_hbm, attr_hbm, ya_hbm, yb_hbm,
             cols_v, idx_v, rows3, out_v, gsems, osems):
        cid = lax.axis_index("c")
        sid = lax.axis_index("s")
        wid = cid * 16 + sid
        tabs = (u0_hbm, u1_hbm, t3_hbm)

        for k in range(5):
            pltpu.sync_copy(attr_hbm.at[k, pl.ds(wid * EPT, EPT)],
                            cols_v.at[k])

        def build_idx(ch, buf):
            for j in range(CHUNK // 16):
                sl = pl.ds(ch * CHUNK + j * 16, 16)
                dst = pl.ds(j * 16, 16)
                idx_v[buf, 0, dst] = cols_v[0, sl] * VPAD + cols_v[1, sl]
                idx_v[buf, 1, dst] = cols_v[2, sl] * VPAD + cols_v[3, sl]
                idx_v[buf, 2, dst] = cols_v[4, sl]

        def gather_copies(buf):
            return [
                pltpu.make_async_copy(
                    tabs[q].at[idx_v.at[buf, q]],
                    rows3.at[buf, pl.ds(q * CHUNK, CHUNK)],
                    gsems[buf])
                for q in range(3)
            ]

        def out_slice(y_hbm, ch):
            return y_hbm.at[pl.ds(sid * (EPT // 2) + ch * ORPC, ORPC)]

        def per_core(fn):
            def _a():
                fn(ya_hbm)
            def _b():
                fn(yb_hbm)
            pl.when(cid == 0)(_a)
            pl.when(cid == 1)(_b)

        def fire_out(ch, buf):
            per_core(lambda y: pltpu.async_copy(
                out_v.at[buf], out_slice(y, ch), osems[buf]))

        def wait_out(buf):
            per_core(lambda y: pltpu.make_async_copy(
                out_v.at[buf], out_slice(y, 0), osems[buf]).wait())

        def process(ch, buf, reclaim):
            if reclaim is None:
                pl.when(ch >= NBUF)(lambda: wait_out(buf))
            elif reclaim:
                wait_out(buf)
            build_idx(ch, buf)
            for cp in gather_copies(buf):
                cp.start()

        def finish(ch, buf):
            for cp in gather_copies(buf):
                cp.wait()

            def esum(m, c2):
                for half in range(2):
                    e = 2 * m + half
                    for cc in range(HIDDEN // 16):
                        sl = pl.ds(cc * 16, 16)
                        dst = pl.ds(half * HIDDEN + cc * 16, 16)
                        out_v[buf, m, dst] = (rows3[buf, e, sl]
                                              + rows3[buf, CHUNK + e, sl]
                                              + rows3[buf, 2 * CHUNK + e, sl])
                return c2

            lax.fori_loop(0, ORPC, esum, 0, unroll=2)
            fire_out(ch, buf)

        process(0, 0, False)
        process(1, 1, False)
        process(2, 2, False)
        finish(0, 0)

        def triple(ti, carry):
            ch = 3 * ti
            for r in range(3):
                process(ch + r, r, None)
                finish(ch + r - 2, (r + 1) % 3)
            return carry

        lax.fori_loop(1, (NCHUNK - 2) // 3, triple, 0)
        process(NCHUNK - 2, 0, True)
        finish(NCHUNK - 4, 1)
        process(NCHUNK - 1, 1, True)
        finish(NCHUNK - 3, 2)
        finish(NCHUNK - 2, 0)
        finish(NCHUNK - 1, 1)
        for buf in (2, 0, 1):
            wait_out(buf)

    return body(u0, u1, t3, attr_t)


def _ln_body(y, g, b, out):
    x = y[...]
    gg = g[...]
    bb = b[...]
    for half in range(2):
        h = x[:, half * HIDDEN:(half + 1) * HIDDEN]
        mu = jnp.mean(h, axis=-1, keepdims=True)
        var = jnp.mean((h - mu) * (h - mu), axis=-1, keepdims=True)
        out[:, half * HIDDEN:(half + 1) * HIDDEN] = (
            (h - mu) * lax.rsqrt(var + 1e-5) * gg + bb)


def _edge_ln(y2, g, b):
    blk = 4000
    rows = y2.shape[0]
    return pl.pallas_call(
        _ln_body,
        grid=(rows // blk,),
        in_specs=[
            pl.BlockSpec((blk, 2 * HIDDEN), lambda i: (i, 0)),
            pl.BlockSpec((HIDDEN,), lambda i: (0,)),
            pl.BlockSpec((HIDDEN,), lambda i: (0,)),
        ],
        out_specs=pl.BlockSpec((blk, 2 * HIDDEN), lambda i: (i, 0)),
        out_shape=jax.ShapeDtypeStruct((rows, 2 * HIDDEN), jnp.float32),
    )(y2, g, b)


def _node_body(x, w, b, g, be, out):
    h = jnp.dot(x[...], w[...], preferred_element_type=jnp.float32) + b[...]
    mu = jnp.mean(h, axis=-1, keepdims=True)
    var = jnp.mean((h - mu) * (h - mu), axis=-1, keepdims=True)
    out[...] = (h - mu) * lax.rsqrt(var + 1e-5) * g[...] + be[...]


def _node_transform(x, w, b, g, be):
    blk = 1000
    return pl.pallas_call(
        _node_body,
        grid=(N // blk,),
        in_specs=[
            pl.BlockSpec((blk, D_NODE), lambda i: (i, 0)),
            pl.BlockSpec((D_NODE, HIDDEN), lambda i: (0, 0)),
            pl.BlockSpec((HIDDEN,), lambda i: (0,)),
            pl.BlockSpec((HIDDEN,), lambda i: (0,)),
            pl.BlockSpec((HIDDEN,), lambda i: (0,)),
        ],
        out_specs=pl.BlockSpec((blk, HIDDEN), lambda i: (i, 0)),
        out_shape=jax.ShapeDtypeStruct((N, HIDDEN), jnp.float32),
    )(x, w, b, g, be)


def kernel(x_s, edge_attr_s, W_node, b_node, g_node, beta_node,
           emb0, emb1, emb2, emb3, W_edge, b_edge, g_edge, beta_edge):
    pe = jnp.asarray(_PE_PAD)
    pad = ((0, VPAD - VOCAB), (0, 0))
    tall5 = _build_tables(pe,
                          jnp.pad(emb0, pad), jnp.pad(emb1, pad),
                          jnp.pad(emb2, pad), jnp.pad(emb3, pad),
                          W_edge, b_edge)
    u0 = _pair_table(tall5[0], tall5[1].reshape(VPAD // 2, 2 * HIDDEN))
    u1 = _pair_table(tall5[2], tall5[3].reshape(VPAD // 2, 2 * HIDDEN))
    ya, yb = _edge_gather_sum(u0, u1, tall5[4], _attr_transpose(edge_attr_s))
    h_e = jnp.concatenate(
        [_edge_ln(ya, g_edge, beta_edge), _edge_ln(yb, g_edge, beta_edge)],
        axis=0).reshape(E, HIDDEN)
    h_x = _node_transform(x_s, W_node, b_node, g_node, beta_node)
    return (h_x, h_e)


# speedup vs baseline: 1.3594x; 1.3594x over previous
import functools

import numpy as np
import jax
import jax.numpy as jnp
from jax import lax
from jax.experimental import pallas as pl
from jax.experimental.pallas import tpu as pltpu
from jax.experimental.pallas import tpu_sc as plsc

HIDDEN = 64
NUM_ATTR = 4
VOCAB = 300
VPAD = 304
PAIRP = VPAD * VPAD
E = 320000
N = 10000
D_NODE = 128

def _pe_const():
    p = np.arange(VOCAB, dtype=np.float64)[:, None]
    i = np.arange(0, HIDDEN, 2, dtype=np.float64)
    div = np.exp(-(i * (np.log(10000.0) / HIDDEN)))
    ang = p * div[None, :]
    pe = np.stack([np.sin(ang), np.cos(ang)], axis=-1).reshape(VOCAB, HIDDEN)
    out = np.zeros((VPAD, HIDDEN), dtype=np.float32)
    out[:VOCAB] = pe.astype(np.float32)
    return out

_PE_PAD = _pe_const()


def _tables_body(pe, e0, e1, e2, e3, w, b, out):
    w0 = w[0:64, :]
    w1 = w[64:128, :]
    w2 = w[128:192, :]
    w3 = w[192:256, :]
    ws = w0 + w1 + w2 + w3
    out[0] = jnp.dot(pe[...], ws, preferred_element_type=jnp.float32) + b[...]
    out[1] = jnp.dot(e0[...], w0, preferred_element_type=jnp.float32)
    out[2] = jnp.dot(e1[...], w1, preferred_element_type=jnp.float32)
    out[3] = jnp.dot(e2[...], w2, preferred_element_type=jnp.float32)
    out[4] = jnp.dot(e3[...], w3, preferred_element_type=jnp.float32)


def _build_tables(pe, e0, e1, e2, e3, w_edge, b_edge):
    return pl.pallas_call(
        _tables_body,
        out_shape=jax.ShapeDtypeStruct((5, VPAD, HIDDEN), jnp.float32),
    )(pe, e0, e1, e2, e3, w_edge, b_edge)


_PBLK = 8


def _pair_body(ta, tb, out):
    tpb = jnp.tile(ta[...], (1, 2))
    out[...] = tpb[:, None, :] + tb[...][None, :, :]


def _pair_table(ta, tbr):
    out = pl.pallas_call(
        _pair_body,
        grid=(VPAD // _PBLK,),
        in_specs=[
            pl.BlockSpec((_PBLK, HIDDEN), lambda i: (i, 0)),
            pl.BlockSpec((VPAD // 2, 2 * HIDDEN), lambda i: (0, 0)),
        ],
        out_specs=pl.BlockSpec((_PBLK, VPAD // 2, 2 * HIDDEN),
                               lambda i: (i, 0, 0)),
        out_shape=jax.ShapeDtypeStruct((VPAD, VPAD // 2, 2 * HIDDEN),
                                       jnp.float32),
    )(ta, tbr)
    return out.reshape(PAIRP, HIDDEN)


NW = 32
EPT = E // NW
CHUNK = 80
NCHUNK = EPT // CHUNK
NBUF = 3
ORPC = CHUNK // 2


def _edge_gather_sum(u0, u1, t3, attr_t):
    mesh = plsc.VectorSubcoreMesh(core_axis_name="c", subcore_axis_name="s")

    @functools.partial(
        pl.kernel,
        out_type=jax.ShapeDtypeStruct((E // 2, 2 * HIDDEN), jnp.float32),
        mesh=mesh,
        compiler_params=pltpu.CompilerParams(use_tc_tiling_on_sc=False),
        scratch_types=[
            pltpu.VMEM((5, EPT), jnp.int32),
            pltpu.VMEM((NBUF, 3, CHUNK), jnp.int32),
            pltpu.VMEM((NBUF, 3 * CHUNK, HIDDEN), jnp.float32),
            pltpu.VMEM((NBUF, ORPC, 2 * HIDDEN), jnp.float32),
            [pltpu.SemaphoreType.DMA] * NBUF,
            [pltpu.SemaphoreType.DMA] * NBUF,
        ],
    )
    def body(u0_hbm, u1_hbm, t3_hbm, attr_hbm, y_hbm,
             cols_v, idx_v, rows3, out_v, gsems, osems):
        wid = lax.axis_index("s") * 2 + lax.axis_index("c")
        tabs = (u0_hbm, u1_hbm, t3_hbm)

        for k in range(5):
            pltpu.sync_copy(attr_hbm.at[k, pl.ds(wid * EPT, EPT)],
                            cols_v.at[k])

        def build_idx(ch, buf):
            for j in range(CHUNK // 16):
                sl = pl.ds(ch * CHUNK + j * 16, 16)
                dst = pl.ds(j * 16, 16)
                idx_v[buf, 0, dst] = cols_v[0, sl] * VPAD + cols_v[1, sl]
                idx_v[buf, 1, dst] = cols_v[2, sl] * VPAD + cols_v[3, sl]
                idx_v[buf, 2, dst] = cols_v[4, sl]

        def gather_copies(buf):
            return [
                pltpu.make_async_copy(
                    tabs[q].at[idx_v.at[buf, q]],
                    rows3.at[buf, pl.ds(q * CHUNK, CHUNK)],
                    gsems[buf])
                for q in range(3)
            ]

        def out_slice(ch):
            return y_hbm.at[pl.ds(wid * (EPT // 2) + ch * ORPC, ORPC)]

        def fire_out(ch, buf):
            pltpu.async_copy(out_v.at[buf], out_slice(ch), osems[buf])

        def wait_out(buf):
            pltpu.make_async_copy(out_v.at[buf], out_slice(0),
                                  osems[buf]).wait()

        def process(ch, buf, reclaim):
            if reclaim is None:
                pl.when(ch >= NBUF)(lambda: wait_out(buf))
            elif reclaim:
                wait_out(buf)
            build_idx(ch, buf)
            for cp in gather_copies(buf):
                cp.start()

        def finish(ch, buf):
            for cp in gather_copies(buf):
                cp.wait()

            def esum(m, c2):
                for half in range(2):
                    e = 2 * m + half
                    for cc in range(HIDDEN // 16):
                        sl = pl.ds(cc * 16, 16)
                        dst = pl.ds(half * HIDDEN + cc * 16, 16)
                        out_v[buf, m, dst] = (rows3[buf, e, sl]
                                              + rows3[buf, CHUNK + e, sl]
                                              + rows3[buf, 2 * CHUNK + e, sl])
                return c2

            lax.fori_loop(0, ORPC, esum, 0, unroll=4)
            fire_out(ch, buf)

        process(0, 0, False)
        process(1, 1, False)
        process(2, 2, False)
        finish(0, 0)

        def triple(ti, carry):
            ch = 3 * ti
            for r in range(3):
                process(ch + r, r, None)
                finish(ch + r - 2, (r + 1) % 3)
            return carry

        lax.fori_loop(1, (NCHUNK - 2) // 3, triple, 0)
        process(NCHUNK - 2, 0, True)
        finish(NCHUNK - 4, 1)
        process(NCHUNK - 1, 1, True)
        finish(NCHUNK - 3, 2)
        finish(NCHUNK - 2, 0)
        finish(NCHUNK - 1, 1)
        for buf in (2, 0, 1):
            wait_out(buf)

    return body(u0, u1, t3, attr_t)


def _ln_body(y, g, b, out):
    x = y[...]
    gg = g[...]
    bb = b[...]
    for half in range(2):
        h = x[:, half * HIDDEN:(half + 1) * HIDDEN]
        mu = jnp.mean(h, axis=-1, keepdims=True)
        var = jnp.mean((h - mu) * (h - mu), axis=-1, keepdims=True)
        out[:, half * HIDDEN:(half + 1) * HIDDEN] = (
            (h - mu) * lax.rsqrt(var + 1e-5) * gg + bb)


def _edge_ln(y2, g, b):
    blk = 4000
    rows = y2.shape[0]
    return pl.pallas_call(
        _ln_body,
        grid=(rows // blk,),
        in_specs=[
            pl.BlockSpec((blk, 2 * HIDDEN), lambda i: (i, 0)),
            pl.BlockSpec((HIDDEN,), lambda i: (0,)),
            pl.BlockSpec((HIDDEN,), lambda i: (0,)),
        ],
        out_specs=pl.BlockSpec((blk, 2 * HIDDEN), lambda i: (i, 0)),
        out_shape=jax.ShapeDtypeStruct((rows, 2 * HIDDEN), jnp.float32),
    )(y2, g, b)


def _node_body(x, w, b, g, be, out):
    h = jnp.dot(x[...], w[...], preferred_element_type=jnp.float32) + b[...]
    mu = jnp.mean(h, axis=-1, keepdims=True)
    var = jnp.mean((h - mu) * (h - mu), axis=-1, keepdims=True)
    out[...] = (h - mu) * lax.rsqrt(var + 1e-5) * g[...] + be[...]


def _node_transform(x, w, b, g, be):
    blk = 1000
    return pl.pallas_call(
        _node_body,
        grid=(N // blk,),
        in_specs=[
            pl.BlockSpec((blk, D_NODE), lambda i: (i, 0)),
            pl.BlockSpec((D_NODE, HIDDEN), lambda i: (0, 0)),
            pl.BlockSpec((HIDDEN,), lambda i: (0,)),
            pl.BlockSpec((HIDDEN,), lambda i: (0,)),
            pl.BlockSpec((HIDDEN,), lambda i: (0,)),
        ],
        out_specs=pl.BlockSpec((blk, HIDDEN), lambda i: (i, 0)),
        out_shape=jax.ShapeDtypeStruct((N, HIDDEN), jnp.float32),
    )(x, w, b, g, be)


def kernel(x_s, edge_attr_s, W_node, b_node, g_node, beta_node,
           emb0, emb1, emb2, emb3, W_edge, b_edge, g_edge, beta_edge):
    pe = jnp.asarray(_PE_PAD)
    pad = ((0, VPAD - VOCAB), (0, 0))
    tall5 = _build_tables(pe,
                          jnp.pad(emb0, pad), jnp.pad(emb1, pad),
                          jnp.pad(emb2, pad), jnp.pad(emb3, pad),
                          W_edge, b_edge)
    u0 = _pair_table(tall5[0], tall5[1].reshape(VPAD // 2, 2 * HIDDEN))
    u1 = _pair_table(tall5[2], tall5[3].reshape(VPAD // 2, 2 * HIDDEN))
    y2 = _edge_gather_sum(u0, u1, tall5[4], edge_attr_s.T)
    h_e = _edge_ln(y2, g_edge, beta_edge).reshape(E, HIDDEN)
    h_x = _node_transform(x_s, W_node, b_node, g_node, beta_node)
    return (h_x, h_e)


# speedup vs baseline: 1.5195x; 1.1177x over previous
import functools

import numpy as np
import jax
import jax.numpy as jnp
from jax import lax
from jax.experimental import pallas as pl
from jax.experimental.pallas import tpu as pltpu
from jax.experimental.pallas import tpu_sc as plsc

HIDDEN = 64
NUM_ATTR = 4
VOCAB = 300
VPAD = 304
PAIRP = VPAD * VPAD
E = 320000
N = 10000
D_NODE = 128

def _pe_const():
    p = np.arange(VOCAB, dtype=np.float64)[:, None]
    i = np.arange(0, HIDDEN, 2, dtype=np.float64)
    div = np.exp(-(i * (np.log(10000.0) / HIDDEN)))
    ang = p * div[None, :]
    pe = np.stack([np.sin(ang), np.cos(ang)], axis=-1).reshape(VOCAB, HIDDEN)
    out = np.zeros((VPAD, HIDDEN), dtype=np.float32)
    out[:VOCAB] = pe.astype(np.float32)
    return out

_PE_PAD = _pe_const()


def _tables_body(pe, e0, e1, e2, e3, w, b, out):
    w0 = w[0:64, :]
    w1 = w[64:128, :]
    w2 = w[128:192, :]
    w3 = w[192:256, :]
    ws = w0 + w1 + w2 + w3
    out[0] = jnp.dot(pe[...], ws, preferred_element_type=jnp.float32) + b[...]
    out[1] = jnp.dot(e0[...], w0, preferred_element_type=jnp.float32)
    out[2] = jnp.dot(e1[...], w1, preferred_element_type=jnp.float32)
    out[3] = jnp.dot(e2[...], w2, preferred_element_type=jnp.float32)
    out[4] = jnp.dot(e3[...], w3, preferred_element_type=jnp.float32)


def _build_tables(pe, e0, e1, e2, e3, w_edge, b_edge):
    return pl.pallas_call(
        _tables_body,
        out_shape=jax.ShapeDtypeStruct((5, VPAD, HIDDEN), jnp.float32),
    )(pe, e0, e1, e2, e3, w_edge, b_edge)


_PBLK = 8


def _pair_body(ta, tb, out):
    tpb = jnp.tile(ta[...], (1, 2))
    out[...] = tpb[:, None, :] + tb[...][None, :, :]


def _pair_table(ta, tbr):
    out = pl.pallas_call(
        _pair_body,
        grid=(VPAD // _PBLK,),
        in_specs=[
            pl.BlockSpec((_PBLK, HIDDEN), lambda i: (i, 0)),
            pl.BlockSpec((VPAD // 2, 2 * HIDDEN), lambda i: (0, 0)),
        ],
        out_specs=pl.BlockSpec((_PBLK, VPAD // 2, 2 * HIDDEN),
                               lambda i: (i, 0, 0)),
        out_shape=jax.ShapeDtypeStruct((VPAD, VPAD // 2, 2 * HIDDEN),
                                       jnp.float32),
    )(ta, tbr)
    return out.reshape(PAIRP, HIDDEN)


NW = 32
EPT = E // NW
CHUNK = 80
NCHUNK = EPT // CHUNK
NBUF = 3
ORPC = CHUNK // 2


def _edge_gather_sum(u0, u1, t3, attr_t):
    mesh = plsc.VectorSubcoreMesh(core_axis_name="c", subcore_axis_name="s")

    @functools.partial(
        pl.kernel,
        out_type=jax.ShapeDtypeStruct((E // 2, 2 * HIDDEN), jnp.float32),
        mesh=mesh,
        compiler_params=pltpu.CompilerParams(use_tc_tiling_on_sc=False),
        scratch_types=[
            pltpu.VMEM((5, EPT), jnp.int32),
            pltpu.VMEM((NBUF, 3, CHUNK), jnp.int32),
            pltpu.VMEM((NBUF, 3 * CHUNK, HIDDEN), jnp.float32),
            pltpu.VMEM((NBUF, CHUNK, HIDDEN), jnp.float32),
            [pltpu.SemaphoreType.DMA] * NBUF,
            [pltpu.SemaphoreType.DMA] * NBUF,
        ],
    )
    def body(u0_hbm, u1_hbm, t3_hbm, attr_hbm, y_hbm,
             cols_v, idx_v, rows3, out_v, gsems, osems):
        wid = lax.axis_index("s") * 2 + lax.axis_index("c")
        tabs = (u0_hbm, u1_hbm, t3_hbm)
        row0 = (wid % 16) * EPT
        loff = (wid // 16) * HIDDEN

        for k in range(5):
            pltpu.sync_copy(attr_hbm.at[k, pl.ds(wid * EPT, EPT)],
                            cols_v.at[k])

        def build_idx(ch, buf):
            for j in range(CHUNK // 16):
                sl = pl.ds(ch * CHUNK + j * 16, 16)
                dst = pl.ds(j * 16, 16)
                idx_v[buf, 0, dst] = cols_v[0, sl] * VPAD + cols_v[1, sl]
                idx_v[buf, 1, dst] = cols_v[2, sl] * VPAD + cols_v[3, sl]
                idx_v[buf, 2, dst] = cols_v[4, sl]

        def gather_copies(buf):
            return [
                pltpu.make_async_copy(
                    tabs[q].at[idx_v.at[buf, q]],
                    rows3.at[buf, pl.ds(q * CHUNK, CHUNK)],
                    gsems[buf])
                for q in range(3)
            ]

        def out_slice(ch):
            return y_hbm.at[pl.ds(row0 + ch * CHUNK, CHUNK),
                            pl.ds(loff, HIDDEN)]

        def fire_out(ch, buf):
            pltpu.async_copy(out_v.at[buf], out_slice(ch), osems[buf])

        def wait_out(buf):
            pltpu.make_async_copy(out_v.at[buf], out_slice(0),
                                  osems[buf]).wait()

        def process(ch, buf, reclaim):
            if reclaim is None:
                pl.when(ch >= NBUF)(lambda: wait_out(buf))
            elif reclaim:
                wait_out(buf)
            build_idx(ch, buf)
            for cp in gather_copies(buf):
                cp.start()

        def finish(ch, buf):
            for cp in gather_copies(buf):
                cp.wait()

            def esum(e, c2):
                for cc in range(HIDDEN // 16):
                    sl = pl.ds(cc * 16, 16)
                    out_v[buf, e, sl] = (rows3[buf, e, sl]
                                         + rows3[buf, CHUNK + e, sl]
                                         + rows3[buf, 2 * CHUNK + e, sl])
                return c2

            lax.fori_loop(0, CHUNK, esum, 0, unroll=8)
            fire_out(ch, buf)

        process(0, 0, False)
        process(1, 1, False)
        process(2, 2, False)
        finish(0, 0)

        def triple(ti, carry):
            ch = 3 * ti
            for r in range(3):
                process(ch + r, r, None)
                finish(ch + r - 2, (r + 1) % 3)
            return carry

        lax.fori_loop(1, (NCHUNK - 2) // 3, triple, 0)
        process(NCHUNK - 2, 0, True)
        finish(NCHUNK - 4, 1)
        process(NCHUNK - 1, 1, True)
        finish(NCHUNK - 3, 2)
        finish(NCHUNK - 2, 0)
        finish(NCHUNK - 1, 1)
        for buf in (2, 0, 1):
            wait_out(buf)

    return body(u0, u1, t3, attr_t)


_LNB = 4000


def _ln_body(y, g, b, out):
    x = y[...]
    h = jnp.where(pl.program_id(0) == 0,
                  x[:, :HIDDEN], x[:, HIDDEN:])
    mu = jnp.mean(h, axis=-1, keepdims=True)
    var = jnp.mean((h - mu) * (h - mu), axis=-1, keepdims=True)
    out[...] = (h - mu) * lax.rsqrt(var + 1e-5) * g[...] + b[...]


def _edge_ln(y2, g, b):
    nblk = E // 2 // _LNB
    return pl.pallas_call(
        _ln_body,
        grid=(2, nblk),
        in_specs=[
            pl.BlockSpec((_LNB, 2 * HIDDEN), lambda h, i: (i, 0)),
            pl.BlockSpec((HIDDEN,), lambda h, i: (0,)),
            pl.BlockSpec((HIDDEN,), lambda h, i: (0,)),
        ],
        out_specs=pl.BlockSpec((_LNB, HIDDEN),
                               lambda h, i: (h * nblk + i, 0)),
        out_shape=jax.ShapeDtypeStruct((E, HIDDEN), jnp.float32),
    )(y2, g, b)


def _node_body(x, w, b, g, be, out):
    h = jnp.dot(x[...], w[...], preferred_element_type=jnp.float32) + b[...]
    mu = jnp.mean(h, axis=-1, keepdims=True)
    var = jnp.mean((h - mu) * (h - mu), axis=-1, keepdims=True)
    out[...] = (h - mu) * lax.rsqrt(var + 1e-5) * g[...] + be[...]


def _node_transform(x, w, b, g, be):
    blk = 1000
    return pl.pallas_call(
        _node_body,
        grid=(N // blk,),
        in_specs=[
            pl.BlockSpec((blk, D_NODE), lambda i: (i, 0)),
            pl.BlockSpec((D_NODE, HIDDEN), lambda i: (0, 0)),
            pl.BlockSpec((HIDDEN,), lambda i: (0,)),
            pl.BlockSpec((HIDDEN,), lambda i: (0,)),
            pl.BlockSpec((HIDDEN,), lambda i: (0,)),
        ],
        out_specs=pl.BlockSpec((blk, HIDDEN), lambda i: (i, 0)),
        out_shape=jax.ShapeDtypeStruct((N, HIDDEN), jnp.float32),
    )(x, w, b, g, be)


def kernel(x_s, edge_attr_s, W_node, b_node, g_node, beta_node,
           emb0, emb1, emb2, emb3, W_edge, b_edge, g_edge, beta_edge):
    pe = jnp.asarray(_PE_PAD)
    pad = ((0, VPAD - VOCAB), (0, 0))
    tall5 = _build_tables(pe,
                          jnp.pad(emb0, pad), jnp.pad(emb1, pad),
                          jnp.pad(emb2, pad), jnp.pad(emb3, pad),
                          W_edge, b_edge)
    u0 = _pair_table(tall5[0], tall5[1].reshape(VPAD // 2, 2 * HIDDEN))
    u1 = _pair_table(tall5[2], tall5[3].reshape(VPAD // 2, 2 * HIDDEN))
    y2 = _edge_gather_sum(u0, u1, tall5[4], edge_attr_s.T)
    h_e = _edge_ln(y2, g_edge, beta_edge)
    h_x = _node_transform(x_s, W_node, b_node, g_node, beta_node)
    return (h_x, h_e)
